# Initial kernel scaffold; baseline (speedup 1.0000x reference)
import jax
import jax.numpy as jnp
from jax.experimental import pallas as pl


def kernel(features, edge_index, W1, b1, W2, b2, D1, bd1, D2, bd2):
    raise NotImplementedError("write your pallas kernel here")


# speedup vs baseline: 6.5146x; 6.5146x over previous
import functools

import jax
import jax.numpy as jnp
from jax import lax
from jax.experimental import pallas as pl
from jax.experimental.pallas import tpu as pltpu
from jax.experimental.pallas import tpu_sc as plsc

N = 10000
NP = 10240
E = 320000
NC, NS = 2, 16
NW = NC * NS
EPW = E // NW
CH = 40
NCHUNK = EPW // CH
NBUF = 5
NGROUP = NCHUNK // NBUF
RPS = NP // NS
DAUG = 144


def _make_sc_agg(d):
  mesh = plsc.VectorSubcoreMesh(core_axis_name="c", subcore_axis_name="s")

  @functools.partial(
      pl.kernel,
      out_type=jax.ShapeDtypeStruct((NC, NP, d), jnp.float32),
      mesh=mesh,
      compiler_params=pltpu.CompilerParams(use_tc_tiling_on_sc=False),
      scratch_types=[
          pltpu.VMEM((NBUF, CH), jnp.int32),
          pltpu.VMEM((NBUF, CH), jnp.int32),
          [pltpu.VMEM((CH, d), jnp.float32) for _ in range(NBUF)],
          pltpu.VMEM_SHARED((NP, d), jnp.float32),
          pltpu.SemaphoreType.DMA,
          pltpu.SemaphoreType.DMA,
      ],
  )
  def agg(h_hbm, src_hbm, dst_hbm, zeros_hbm, out_hbm,
          src_v, dst_v, rows, acc, gsem, ssem):
    c = lax.axis_index("c")
    s = lax.axis_index("s")
    w = s * NC + c
    pltpu.sync_copy(zeros_hbm.at[pl.ds(s * RPS, RPS)],
                    acc.at[pl.ds(s * RPS, RPS)])
    plsc.subcore_barrier()

    def group(g, carry):
      pltpu.sync_copy(src_hbm.at[w].at[pl.ds(g * NBUF, NBUF)], src_v)
      pltpu.sync_copy(dst_hbm.at[w].at[pl.ds(g * NBUF, NBUF)], dst_v)
      gathers = []
      for b in range(NBUF):
        gathers.append(
            pltpu.async_copy(h_hbm.at[src_v.at[b]], rows[b], gsem))
      for cp in gathers:
        cp.wait()
      scatters = []
      for b in range(NBUF):
        scatters.append(
            pltpu.async_copy(rows[b], acc.at[dst_v.at[b]], ssem,
                             add=True))
      for cp in scatters:
        cp.wait()
      return carry

    lax.fori_loop(0, NGROUP, group, 0)
    plsc.subcore_barrier()
    pltpu.sync_copy(acc.at[pl.ds(s * RPS, RPS)],
                    out_hbm.at[c].at[pl.ds(s * RPS, RPS)])

  return agg


_sc_agg_aug = _make_sc_agg(DAUG)
_sc_agg_128 = _make_sc_agg(128)

_R = 1024


def _tc1_body(accp, feat, w1, b1, h1_out, rinv_out):
  hsum = accp[0] + accp[1]
  agg = hsum[:, :128]
  deg = hsum[:, 128:129]
  rinv = 1.0 / (deg + 1.0)
  hn = (agg + feat[...]) * rinv
  h1 = jnp.dot(hn, w1[...], preferred_element_type=jnp.float32) + b1[...]
  h1_out[...] = jnp.maximum(h1, 0.0)
  rinv_out[...] = jnp.broadcast_to(rinv, (_R, 128))


_tc1 = pl.pallas_call(
    _tc1_body,
    grid=(NP // _R,),
    in_specs=[
        pl.BlockSpec((NC, _R, DAUG), lambda i: (0, i, 0)),
        pl.BlockSpec((_R, 128), lambda i: (i, 0)),
        pl.BlockSpec((128, 128), lambda i: (0, 0)),
        pl.BlockSpec((1, 128), lambda i: (0, 0)),
    ],
    out_specs=[
        pl.BlockSpec((_R, 128), lambda i: (i, 0)),
        pl.BlockSpec((_R, 128), lambda i: (i, 0)),
    ],
    out_shape=[
        jax.ShapeDtypeStruct((NP, 128), jnp.float32),
        jax.ShapeDtypeStruct((NP, 128), jnp.float32),
    ],
)


def _leaky(x):
  return jnp.where(x > 0, x, 0.01 * x)


def _tc2_body(accp, h1, rinv, w2, b2, d1, bd1, d2, bd2, out):
  hsum = accp[0] + accp[1]
  hn = (hsum + h1[...]) * rinv[:, :1]
  h2 = jnp.dot(hn, w2[...], preferred_element_type=jnp.float32) + b2[...]
  h2 = jnp.maximum(h2, 0.0)
  t = jnp.dot(h2, d1[...], preferred_element_type=jnp.float32) + bd1[...]
  t = _leaky(t)
  o = jnp.dot(t, d2[...], preferred_element_type=jnp.float32) + bd2[...]
  out[...] = _leaky(o)


_tc2 = pl.pallas_call(
    _tc2_body,
    grid=(NP // _R,),
    in_specs=[
        pl.BlockSpec((NC, _R, 128), lambda i: (0, i, 0)),
        pl.BlockSpec((_R, 128), lambda i: (i, 0)),
        pl.BlockSpec((_R, 128), lambda i: (i, 0)),
        pl.BlockSpec((128, 128), lambda i: (0, 0)),
        pl.BlockSpec((1, 128), lambda i: (0, 0)),
        pl.BlockSpec((128, 256), lambda i: (0, 0)),
        pl.BlockSpec((1, 256), lambda i: (0, 0)),
        pl.BlockSpec((256, 40), lambda i: (0, 0)),
        pl.BlockSpec((1, 40), lambda i: (0, 0)),
    ],
    out_specs=pl.BlockSpec((_R, 40), lambda i: (i, 0)),
    out_shape=jax.ShapeDtypeStruct((NP, 40), jnp.float32),
)


@jax.jit
def kernel(features, edge_index, W1, b1, W2, b2, D1, bd1, D2, bd2):
  ei = edge_index.astype(jnp.int32)
  src = ei[0].reshape(NW, NCHUNK, CH)
  dst = ei[1].reshape(NW, NCHUNK, CH)
  featp = jnp.pad(features, ((0, NP - N), (0, 0)))
  aug = jnp.pad(featp, ((0, 0), (0, DAUG - 128)))
  aug = aug.at[:, 128].set(1.0)
  acc1 = _sc_agg_aug(aug, src, dst, jnp.zeros((NP, DAUG), jnp.float32))
  h1, rinv = _tc1(acc1, featp, W1, b1.reshape(1, 128))
  acc2 = _sc_agg_128(h1, src, dst, jnp.zeros((NP, 128), jnp.float32))
  out = _tc2(acc2, h1, rinv, W2, b2.reshape(1, 128),
             D1, bd1.reshape(1, 256), D2, bd2.reshape(1, 40))
  return out[:N]


# speedup vs baseline: 8.5599x; 1.3140x over previous
import functools

import jax
import jax.numpy as jnp
from jax import lax
from jax.experimental import pallas as pl
from jax.experimental.pallas import tpu as pltpu
from jax.experimental.pallas import tpu_sc as plsc

N = 10000
NP = 10240
E = 320000
NC, NS = 2, 16
NW = NC * NS
EPW = E // NW
CH = 40
NCHUNK = EPW // CH
NBUF = 5
NGROUP = NCHUNK // NBUF
RPS = NP // NS
DAUG = 144


def _make_sc_agg(d):
  mesh = plsc.VectorSubcoreMesh(core_axis_name="c", subcore_axis_name="s")

  @functools.partial(
      pl.kernel,
      out_type=jax.ShapeDtypeStruct((NC, NP, d), jnp.float32),
      mesh=mesh,
      compiler_params=pltpu.CompilerParams(use_tc_tiling_on_sc=False),
      scratch_types=[
          [pltpu.VMEM((NBUF, CH), jnp.int32) for _ in range(2)],
          [pltpu.VMEM((NBUF, CH), jnp.int32) for _ in range(2)],
          [pltpu.VMEM((CH, d), jnp.float32) for _ in range(NBUF)],
          pltpu.VMEM_SHARED((NP, d), jnp.float32),
          [pltpu.SemaphoreType.DMA for _ in range(NBUF)],
          [pltpu.SemaphoreType.DMA for _ in range(NBUF)],
      ],
  )
  def agg(h_hbm, src_hbm, dst_hbm, zeros_hbm, out_hbm,
          src_v, dst_v, rows, acc, gsem, ssem):
    c = lax.axis_index("c")
    s = lax.axis_index("s")
    w = s * NC + c

    def stage_idx(g, p):
      pltpu.sync_copy(src_hbm.at[w].at[pl.ds(g * NBUF, NBUF)], src_v[p])
      pltpu.sync_copy(dst_hbm.at[w].at[pl.ds(g * NBUF, NBUF)], dst_v[p])

    def issue_gather(b, p):
      pltpu.async_copy(h_hbm.at[src_v[p].at[b]], rows[b], gsem[b])

    def step(g, p, stage_next, issue_next):
      if stage_next:
        stage_idx(g + 1, 1 - p)
      for b in range(NBUF):
        pltpu.make_async_copy(h_hbm.at[src_v[p].at[b]], rows[b],
                              gsem[b]).wait()
        pltpu.async_copy(rows[b], acc.at[dst_v[p].at[b]], ssem[b], add=True)
      for b in range(NBUF):
        pltpu.make_async_copy(rows[b], acc.at[dst_v[p].at[b]], ssem[b]).wait()
        if issue_next:
          issue_gather(b, 1 - p)

    pltpu.sync_copy(zeros_hbm.at[pl.ds(s * RPS, RPS)],
                    acc.at[pl.ds(s * RPS, RPS)])
    stage_idx(0, 0)
    for b in range(NBUF):
      issue_gather(b, 0)
    plsc.subcore_barrier()

    def pair(k, carry):
      step(2 * k, 0, True, True)
      step(2 * k + 1, 1, True, True)
      return carry

    lax.fori_loop(0, NGROUP // 2 - 1, pair, 0)
    step(NGROUP - 2, 0, True, True)
    step(NGROUP - 1, 1, False, False)
    plsc.subcore_barrier()
    pltpu.sync_copy(acc.at[pl.ds(s * RPS, RPS)],
                    out_hbm.at[c].at[pl.ds(s * RPS, RPS)])

  return agg


_sc_agg_aug = _make_sc_agg(DAUG)
_sc_agg_128 = _make_sc_agg(128)

_R = 1024


def _tc1_body(accp, feat, w1, b1, h1_out, rinv_out):
  hsum = accp[0] + accp[1]
  agg = hsum[:, :128]
  deg = hsum[:, 128:129]
  rinv = 1.0 / (deg + 1.0)
  hn = (agg + feat[...]) * rinv
  h1 = jnp.dot(hn, w1[...], preferred_element_type=jnp.float32) + b1[...]
  h1_out[...] = jnp.maximum(h1, 0.0)
  rinv_out[...] = jnp.broadcast_to(rinv, (_R, 128))


_tc1 = pl.pallas_call(
    _tc1_body,
    grid=(NP // _R,),
    in_specs=[
        pl.BlockSpec((NC, _R, DAUG), lambda i: (0, i, 0)),
        pl.BlockSpec((_R, 128), lambda i: (i, 0)),
        pl.BlockSpec((128, 128), lambda i: (0, 0)),
        pl.BlockSpec((1, 128), lambda i: (0, 0)),
    ],
    out_specs=[
        pl.BlockSpec((_R, 128), lambda i: (i, 0)),
        pl.BlockSpec((_R, 128), lambda i: (i, 0)),
    ],
    out_shape=[
        jax.ShapeDtypeStruct((NP, 128), jnp.float32),
        jax.ShapeDtypeStruct((NP, 128), jnp.float32),
    ],
)


def _leaky(x):
  return jnp.where(x > 0, x, 0.01 * x)


def _tc2_body(accp, h1, rinv, w2, b2, d1, bd1, d2, bd2, out):
  hsum = accp[0] + accp[1]
  hn = (hsum + h1[...]) * rinv[:, :1]
  h2 = jnp.dot(hn, w2[...], preferred_element_type=jnp.float32) + b2[...]
  h2 = jnp.maximum(h2, 0.0)
  t = jnp.dot(h2, d1[...], preferred_element_type=jnp.float32) + bd1[...]
  t = _leaky(t)
  o = jnp.dot(t, d2[...], preferred_element_type=jnp.float32) + bd2[...]
  out[...] = _leaky(o)


_tc2 = pl.pallas_call(
    _tc2_body,
    grid=(NP // _R,),
    in_specs=[
        pl.BlockSpec((NC, _R, 128), lambda i: (0, i, 0)),
        pl.BlockSpec((_R, 128), lambda i: (i, 0)),
        pl.BlockSpec((_R, 128), lambda i: (i, 0)),
        pl.BlockSpec((128, 128), lambda i: (0, 0)),
        pl.BlockSpec((1, 128), lambda i: (0, 0)),
        pl.BlockSpec((128, 256), lambda i: (0, 0)),
        pl.BlockSpec((1, 256), lambda i: (0, 0)),
        pl.BlockSpec((256, 40), lambda i: (0, 0)),
        pl.BlockSpec((1, 40), lambda i: (0, 0)),
    ],
    out_specs=pl.BlockSpec((_R, 40), lambda i: (i, 0)),
    out_shape=jax.ShapeDtypeStruct((NP, 40), jnp.float32),
)


@jax.jit
def kernel(features, edge_index, W1, b1, W2, b2, D1, bd1, D2, bd2):
  ei = edge_index.astype(jnp.int32)
  src = ei[0].reshape(NW, NCHUNK, CH)
  dst = ei[1].reshape(NW, NCHUNK, CH)
  featp = jnp.pad(features, ((0, NP - N), (0, 0)))
  aug = jnp.pad(featp, ((0, 0), (0, DAUG - 128)))
  aug = aug.at[:, 128].set(1.0)
  acc1 = _sc_agg_aug(aug, src, dst, jnp.zeros((NP, DAUG), jnp.float32))
  h1, rinv = _tc1(acc1, featp, W1, b1.reshape(1, 128))
  acc2 = _sc_agg_128(h1, src, dst, jnp.zeros((NP, 128), jnp.float32))
  out = _tc2(acc2, h1, rinv, W2, b2.reshape(1, 128),
             D1, bd1.reshape(1, 256), D2, bd2.reshape(1, 40))
  return out[:N]


# speedup vs baseline: 9.6619x; 1.1287x over previous
import functools

import jax
import jax.numpy as jnp
from jax import lax
from jax.experimental import pallas as pl
from jax.experimental.pallas import tpu as pltpu
from jax.experimental.pallas import tpu_sc as plsc

N = 10000
NP = 10240
E = 320000
NC, NS = 2, 16
NW = NC * NS
EPW = E // NW
CH = 40
NCHUNK = EPW // CH
NBUF = 5
NGROUP = NCHUNK // NBUF
RPS = NP // NS
DAUG = 144


def _make_sc_agg(d):
  mesh = plsc.VectorSubcoreMesh(core_axis_name="c", subcore_axis_name="s")

  @functools.partial(
      pl.kernel,
      out_type=jax.ShapeDtypeStruct((NC, NP, d), jnp.float32),
      mesh=mesh,
      compiler_params=pltpu.CompilerParams(use_tc_tiling_on_sc=False),
      scratch_types=[
          [pltpu.VMEM((NBUF, CH), jnp.int32) for _ in range(2)],
          [pltpu.VMEM((NBUF, CH), jnp.int32) for _ in range(2)],
          [pltpu.VMEM((CH, d), jnp.float32) for _ in range(NBUF)],
          pltpu.VMEM_SHARED((NP, d), jnp.float32),
          [pltpu.SemaphoreType.DMA for _ in range(NBUF)],
          [pltpu.SemaphoreType.DMA for _ in range(NBUF)],
          [pltpu.SemaphoreType.DMA for _ in range(2)],
      ],
  )
  def agg(h_hbm, src_hbm, dst_hbm, zeros_hbm, out_hbm,
          src_v, dst_v, rows, acc, gsem, ssem, isem):
    c = lax.axis_index("c")
    s = lax.axis_index("s")
    w = s * NC + c

    def stage_idx(g, p):
      pltpu.async_copy(src_hbm.at[w].at[pl.ds(g * NBUF, NBUF)], src_v[p],
                       isem[p])
      pltpu.async_copy(dst_hbm.at[w].at[pl.ds(g * NBUF, NBUF)], dst_v[p],
                       isem[p])

    def wait_idx(g, p):
      pltpu.make_async_copy(src_hbm.at[w].at[pl.ds(g * NBUF, NBUF)],
                            src_v[p], isem[p]).wait()
      pltpu.make_async_copy(dst_hbm.at[w].at[pl.ds(g * NBUF, NBUF)],
                            dst_v[p], isem[p]).wait()

    def issue_gather(b, p):
      pltpu.async_copy(h_hbm.at[src_v[p].at[b]], rows[b], gsem[b])

    def step(g, p, stage_next, issue_next):
      if stage_next:
        stage_idx(g + 1, 1 - p)
      for b in range(NBUF):
        pltpu.make_async_copy(h_hbm.at[src_v[p].at[b]], rows[b],
                              gsem[b]).wait()
        pltpu.async_copy(rows[b], acc.at[dst_v[p].at[b]], ssem[b], add=True)
      if issue_next:
        wait_idx(g + 1, 1 - p)
      for b in range(NBUF):
        pltpu.make_async_copy(rows[b], acc.at[dst_v[p].at[b]], ssem[b]).wait()
        if issue_next:
          issue_gather(b, 1 - p)

    pltpu.sync_copy(zeros_hbm.at[pl.ds(s * RPS, RPS)],
                    acc.at[pl.ds(s * RPS, RPS)])
    stage_idx(0, 0)
    wait_idx(0, 0)
    for b in range(NBUF):
      issue_gather(b, 0)
    plsc.subcore_barrier()

    def pair(k, carry):
      step(2 * k, 0, True, True)
      step(2 * k + 1, 1, True, True)
      return carry

    lax.fori_loop(0, NGROUP // 2 - 1, pair, 0)
    step(NGROUP - 2, 0, True, True)
    step(NGROUP - 1, 1, False, False)
    plsc.subcore_barrier()
    pltpu.sync_copy(acc.at[pl.ds(s * RPS, RPS)],
                    out_hbm.at[c].at[pl.ds(s * RPS, RPS)])

  return agg


_sc_agg_aug = _make_sc_agg(DAUG)
_sc_agg_128 = _make_sc_agg(128)

_R = 1024


def _tc1_body(accp, feat, w1, b1, h1_out, rinv_out):
  hsum = accp[0] + accp[1]
  agg = hsum[:, :128]
  deg = hsum[:, 128:129]
  rinv = 1.0 / (deg + 1.0)
  hn = (agg + feat[...]) * rinv
  h1 = jnp.dot(hn, w1[...], preferred_element_type=jnp.float32) + b1[...]
  h1_out[...] = jnp.maximum(h1, 0.0)
  rinv_out[...] = jnp.broadcast_to(rinv, (_R, 128))


_tc1 = pl.pallas_call(
    _tc1_body,
    grid=(NP // _R,),
    in_specs=[
        pl.BlockSpec((NC, _R, DAUG), lambda i: (0, i, 0)),
        pl.BlockSpec((_R, 128), lambda i: (i, 0)),
        pl.BlockSpec((128, 128), lambda i: (0, 0)),
        pl.BlockSpec((1, 128), lambda i: (0, 0)),
    ],
    out_specs=[
        pl.BlockSpec((_R, 128), lambda i: (i, 0)),
        pl.BlockSpec((_R, 128), lambda i: (i, 0)),
    ],
    out_shape=[
        jax.ShapeDtypeStruct((NP, 128), jnp.float32),
        jax.ShapeDtypeStruct((NP, 128), jnp.float32),
    ],
)


def _leaky(x):
  return jnp.where(x > 0, x, 0.01 * x)


def _tc2_body(accp, h1, rinv, w2, b2, d1, bd1, d2, bd2, out):
  hsum = accp[0] + accp[1]
  hn = (hsum + h1[...]) * rinv[:, :1]
  h2 = jnp.dot(hn, w2[...], preferred_element_type=jnp.float32) + b2[...]
  h2 = jnp.maximum(h2, 0.0)
  t = jnp.dot(h2, d1[...], preferred_element_type=jnp.float32) + bd1[...]
  t = _leaky(t)
  o = jnp.dot(t, d2[...], preferred_element_type=jnp.float32) + bd2[...]
  out[...] = _leaky(o)


_tc2 = pl.pallas_call(
    _tc2_body,
    grid=(NP // _R,),
    in_specs=[
        pl.BlockSpec((NC, _R, 128), lambda i: (0, i, 0)),
        pl.BlockSpec((_R, 128), lambda i: (i, 0)),
        pl.BlockSpec((_R, 128), lambda i: (i, 0)),
        pl.BlockSpec((128, 128), lambda i: (0, 0)),
        pl.BlockSpec((1, 128), lambda i: (0, 0)),
        pl.BlockSpec((128, 256), lambda i: (0, 0)),
        pl.BlockSpec((1, 256), lambda i: (0, 0)),
        pl.BlockSpec((256, 40), lambda i: (0, 0)),
        pl.BlockSpec((1, 40), lambda i: (0, 0)),
    ],
    out_specs=pl.BlockSpec((_R, 40), lambda i: (i, 0)),
    out_shape=jax.ShapeDtypeStruct((NP, 40), jnp.float32),
)


@jax.jit
def kernel(features, edge_index, W1, b1, W2, b2, D1, bd1, D2, bd2):
  ei = edge_index.astype(jnp.int32)
  src = ei[0].reshape(NW, NCHUNK, CH)
  dst = ei[1].reshape(NW, NCHUNK, CH)
  featp = jnp.pad(features, ((0, NP - N), (0, 0)))
  aug = jnp.pad(featp, ((0, 0), (0, DAUG - 128)))
  aug = aug.at[:, 128].set(1.0)
  acc1 = _sc_agg_aug(aug, src, dst, jnp.zeros((NP, DAUG), jnp.float32))
  h1, rinv = _tc1(acc1, featp, W1, b1.reshape(1, 128))
  acc2 = _sc_agg_128(h1, src, dst, jnp.zeros((NP, 128), jnp.float32))
  out = _tc2(acc2, h1, rinv, W2, b2.reshape(1, 128),
             D1, bd1.reshape(1, 256), D2, bd2.reshape(1, 40))
  return out[:N]
